# Initial kernel scaffold; baseline (speedup 1.0000x reference)
import jax
import jax.numpy as jnp
from jax.experimental import pallas as pl


def kernel(x, edge_index, eps, W1, b1, bn_gamma, bn_beta, W2, b2):
    raise NotImplementedError("write your pallas kernel here")


# speedup vs baseline: 6.0776x; 6.0776x over previous
import functools

import jax
import jax.numpy as jnp
from jax import lax
from jax.experimental import pallas as pl
from jax.experimental.pallas import tpu as pltpu
from jax.experimental.pallas import tpu_sc as plsc

N = 10000
E = 320000
D = 128

NC = 2
NS = 16
NW = NC * NS

EPT = E // NW
CH = 128
NFULL = EPT // CH
TAIL = EPT - NFULL * CH
ZR = (N // NS) // 8 * 8
ZTAIL = N - NS * ZR


@functools.partial(
    pl.kernel,
    out_type=jax.ShapeDtypeStruct((NC, N, D), jnp.float32),
    mesh=plsc.VectorSubcoreMesh(core_axis_name="c", subcore_axis_name="s"),
    scratch_types=[
        pltpu.VMEM((CH,), jnp.int32),
        pltpu.VMEM((CH,), jnp.int32),
        pltpu.VMEM((CH, D), jnp.float32),
        pltpu.VMEM((TAIL,), jnp.int32),
        pltpu.VMEM((TAIL,), jnp.int32),
        pltpu.VMEM((TAIL, D), jnp.float32),
        pltpu.VMEM_SHARED((N, D), jnp.float32),
        pltpu.SemaphoreType.DMA,
    ],
)
def _sc_aggregate(x_hbm, row_hbm, col_hbm, zero_hbm, out_hbm,
                  row_v, col_v, rows_v, rowt_v, colt_v, rowst_v, aggr_sh, sem):
    cid = lax.axis_index("c")
    sid = lax.axis_index("s")
    wid = cid * NS + sid

    pltpu.sync_copy(zero_hbm.at[pl.ds(sid * ZR, ZR)],
                    aggr_sh.at[pl.ds(sid * ZR, ZR)])

    @pl.when(sid == NS - 1)
    def _():
        pltpu.sync_copy(zero_hbm.at[pl.ds(NS * ZR, ZTAIL)],
                        aggr_sh.at[pl.ds(NS * ZR, ZTAIL)])

    plsc.subcore_barrier()

    base = wid * EPT

    def body(j, carry):
        off = base + j * CH
        pltpu.sync_copy(row_hbm.at[pl.ds(off, CH)], row_v)
        pltpu.sync_copy(col_hbm.at[pl.ds(off, CH)], col_v)
        pltpu.async_copy(x_hbm.at[row_v], rows_v, sem).wait()
        pltpu.sync_copy(rows_v, aggr_sh.at[col_v], add=True)
        return carry

    lax.fori_loop(0, NFULL, body, 0)

    off = base + NFULL * CH
    pltpu.sync_copy(row_hbm.at[pl.ds(off, TAIL)], rowt_v)
    pltpu.sync_copy(col_hbm.at[pl.ds(off, TAIL)], colt_v)
    pltpu.async_copy(x_hbm.at[rowt_v], rowst_v, sem).wait()
    pltpu.sync_copy(rowst_v, aggr_sh.at[colt_v], add=True)

    plsc.subcore_barrier()

    pltpu.sync_copy(aggr_sh.at[pl.ds(sid * ZR, ZR)],
                    out_hbm.at[cid, pl.ds(sid * ZR, ZR)])

    @pl.when(sid == NS - 1)
    def _():
        pltpu.sync_copy(aggr_sh.at[pl.ds(NS * ZR, ZTAIL)],
                        out_hbm.at[cid, pl.ds(NS * ZR, ZTAIL)])


def _mlp_body(eps_ref, x_ref, a_ref, w1_ref, b1_ref, g_ref, be_ref,
              w2_ref, b2_ref, o_ref):
    out = x_ref[...] * (1.0 + eps_ref[0]) + a_ref[0] + a_ref[1]
    h = lax.dot_general(out, w1_ref[...], (((1,), (1,)), ((), ())),
                        preferred_element_type=jnp.float32) + b1_ref[...]
    mu = jnp.mean(h, axis=0, keepdims=True)
    c = h - mu
    var = jnp.mean(c * c, axis=0, keepdims=True)
    hn = c * lax.rsqrt(var + 1e-5) * g_ref[...] + be_ref[...]
    hn = jnp.maximum(hn, 0.0)
    o_ref[...] = lax.dot_general(hn, w2_ref[...], (((1,), (1,)), ((), ())),
                                 preferred_element_type=jnp.float32) + b2_ref[...]


_mlp = pl.pallas_call(
    _mlp_body,
    out_shape=jax.ShapeDtypeStruct((N, D), jnp.float32),
    in_specs=[
        pl.BlockSpec(memory_space=pltpu.SMEM),
        pl.BlockSpec(memory_space=pltpu.VMEM),
        pl.BlockSpec(memory_space=pltpu.VMEM),
        pl.BlockSpec(memory_space=pltpu.VMEM),
        pl.BlockSpec(memory_space=pltpu.VMEM),
        pl.BlockSpec(memory_space=pltpu.VMEM),
        pl.BlockSpec(memory_space=pltpu.VMEM),
        pl.BlockSpec(memory_space=pltpu.VMEM),
        pl.BlockSpec(memory_space=pltpu.VMEM),
    ],
    out_specs=pl.BlockSpec(memory_space=pltpu.VMEM),
)


def kernel(x, edge_index, eps, W1, b1, bn_gamma, bn_beta, W2, b2):
    ei = edge_index.astype(jnp.int32)
    row = ei[0]
    col = ei[1]
    zeros = jnp.zeros((N, D), jnp.float32)
    aggr = _sc_aggregate(x, row, col, zeros)
    return _mlp(eps, x, aggr, W1, b1.reshape(1, D), bn_gamma.reshape(1, D),
                bn_beta.reshape(1, D), W2, b2.reshape(1, D))


# speedup vs baseline: 11.0344x; 1.8156x over previous
import functools

import jax
import jax.numpy as jnp
from jax import lax
from jax.experimental import pallas as pl
from jax.experimental.pallas import tpu as pltpu
from jax.experimental.pallas import tpu_sc as plsc

N = 10000
E = 320000
D = 128

NC = 2
NS = 16
NW = NC * NS

EPT = E // NW
CH = 125
NCH = EPT // CH
PCH = 40
PH = NCH // PCH
ZR = (N // NS) // 8 * 8
ZTAIL = N - NS * ZR


@functools.partial(
    pl.kernel,
    out_type=jax.ShapeDtypeStruct((NC, N, D), jnp.float32),
    mesh=plsc.VectorSubcoreMesh(core_axis_name="c", subcore_axis_name="s"),
    scratch_types=[
        pltpu.VMEM((PCH, CH), jnp.int32),
        pltpu.VMEM((PCH, CH), jnp.int32),
        pltpu.VMEM((CH, D), jnp.float32),
        pltpu.VMEM((CH, D), jnp.float32),
        pltpu.VMEM_SHARED((N, D), jnp.float32),
        pltpu.SemaphoreType.DMA,
        pltpu.SemaphoreType.DMA,
        pltpu.SemaphoreType.DMA,
        pltpu.SemaphoreType.DMA,
    ],
)
def _sc_aggregate(x_hbm, row_hbm, col_hbm, zero_hbm, out_hbm,
                  row_v, col_v, bufa, bufb, aggr_sh,
                  gsa, gsb, ssa, ssb):
    cid = lax.axis_index("c")
    sid = lax.axis_index("s")
    wid = cid * NS + sid

    pltpu.sync_copy(zero_hbm.at[pl.ds(sid * ZR, ZR)],
                    aggr_sh.at[pl.ds(sid * ZR, ZR)])

    @pl.when(sid == NS - 1)
    def _():
        pltpu.sync_copy(zero_hbm.at[pl.ds(NS * ZR, ZTAIL)],
                        aggr_sh.at[pl.ds(NS * ZR, ZTAIL)])

    plsc.subcore_barrier()

    bufs = (bufa, bufb)
    gsems = (gsa, gsb)
    ssems = (ssa, ssb)

    def start_gather(j, b):
        pltpu.async_copy(x_hbm.at[row_v.at[j]], bufs[b], gsems[b])

    def wait_gather(j, b):
        pltpu.make_async_copy(x_hbm.at[row_v.at[j]], bufs[b], gsems[b]).wait()

    def scatter(j, b):
        return pltpu.async_copy(bufs[b], aggr_sh.at[col_v.at[j]], ssems[b],
                                add=True)

    def phase(p, carry):
        pltpu.sync_copy(row_hbm.at[pl.ds(wid * NCH + p * PCH, PCH)], row_v)
        pltpu.sync_copy(col_hbm.at[pl.ds(wid * NCH + p * PCH, PCH)], col_v)

        start_gather(0, 0)
        start_gather(1, 1)

        def body(jj, carry):
            for b in range(2):
                j = 2 * jj + b
                wait_gather(j, b)
                scatter(j, b).wait()
                start_gather(j + 2, b)
            return carry

        lax.fori_loop(0, PCH // 2 - 1, body, 0)

        for b in range(2):
            j = PCH - 2 + b
            wait_gather(j, b)
            scatter(j, b).wait()
        return carry

    lax.fori_loop(0, PH, phase, 0)

    plsc.subcore_barrier()

    pltpu.sync_copy(aggr_sh.at[pl.ds(sid * ZR, ZR)],
                    out_hbm.at[cid, pl.ds(sid * ZR, ZR)])

    @pl.when(sid == NS - 1)
    def _():
        pltpu.sync_copy(aggr_sh.at[pl.ds(NS * ZR, ZTAIL)],
                        out_hbm.at[cid, pl.ds(NS * ZR, ZTAIL)])


def _mlp_body(eps_ref, x_ref, a_ref, w1_ref, b1_ref, g_ref, be_ref,
              w2_ref, b2_ref, o_ref):
    out = x_ref[...] * (1.0 + eps_ref[0]) + a_ref[0] + a_ref[1]
    h = lax.dot_general(out, w1_ref[...], (((1,), (1,)), ((), ())),
                        preferred_element_type=jnp.float32) + b1_ref[...]
    mu = jnp.mean(h, axis=0, keepdims=True)
    c = h - mu
    var = jnp.mean(c * c, axis=0, keepdims=True)
    hn = c * lax.rsqrt(var + 1e-5) * g_ref[...] + be_ref[...]
    hn = jnp.maximum(hn, 0.0)
    o_ref[...] = lax.dot_general(hn, w2_ref[...], (((1,), (1,)), ((), ())),
                                 preferred_element_type=jnp.float32) + b2_ref[...]


_mlp = pl.pallas_call(
    _mlp_body,
    out_shape=jax.ShapeDtypeStruct((N, D), jnp.float32),
    in_specs=[
        pl.BlockSpec(memory_space=pltpu.SMEM),
        pl.BlockSpec(memory_space=pltpu.VMEM),
        pl.BlockSpec(memory_space=pltpu.VMEM),
        pl.BlockSpec(memory_space=pltpu.VMEM),
        pl.BlockSpec(memory_space=pltpu.VMEM),
        pl.BlockSpec(memory_space=pltpu.VMEM),
        pl.BlockSpec(memory_space=pltpu.VMEM),
        pl.BlockSpec(memory_space=pltpu.VMEM),
        pl.BlockSpec(memory_space=pltpu.VMEM),
    ],
    out_specs=pl.BlockSpec(memory_space=pltpu.VMEM),
)


def kernel(x, edge_index, eps, W1, b1, bn_gamma, bn_beta, W2, b2):
    ei = edge_index.astype(jnp.int32)
    row = ei[0].reshape(NW * NCH, CH)
    col = ei[1].reshape(NW * NCH, CH)
    zeros = jnp.zeros((N, D), jnp.float32)
    aggr = _sc_aggregate(x, row, col, zeros)
    return _mlp(eps, x, aggr, W1, b1.reshape(1, D), bn_gamma.reshape(1, D),
                bn_beta.reshape(1, D), W2, b2.reshape(1, D))
